# Initial kernel scaffold; baseline (speedup 1.0000x reference)
import functools

import jax
import jax.numpy as jnp
from jax import lax
from jax.experimental import pallas as pl
from jax.experimental.pallas import tpu as pltpu
from jax.experimental.pallas import tpu_sc as plsc

_NC = 2
_NS = 16
_L = 16

_ROWS_PER_SC = 255
_CH = 3200
_FLUSH = 128


def _impl(left, right, weight, *, interpret=False):
    n_pairs = left.shape[0]
    v = weight.shape[0]
    r = min(_ROWS_PER_SC, v // _NC)
    nu = v // (_NC * r)
    rem = v - nu * _NC * r
    rt = rem // _NC
    assert rem % _NC == 0
    assert n_pairs % _CH == 0 and _CH % _FLUSH == 0 and _FLUSH % _L == 0
    nchunk = n_pairs // _CH
    share = r * v // _NS
    assert r * v % _NS == 0
    if rt:
        assert rt * v % _NS == 0

    mesh = plsc.VectorSubcoreMesh(
        core_axis_name="c", subcore_axis_name="s",
        num_cores=_NC, num_subcores=_NS)

    @functools.partial(
        pl.kernel,
        out_type=jax.ShapeDtypeStruct((v * v,), jnp.float32),
        mesh=mesh,
        scratch_types=[
            pltpu.VMEM_SHARED((r * v,), jnp.float32),
            pltpu.VMEM((_CH,), jnp.int32),
            pltpu.VMEM((_CH,), jnp.int32),
            pltpu.VMEM((_FLUSH,), jnp.int32),
            pltpu.VMEM((_FLUSH,), jnp.float32),
        ],
        interpret=interpret,
    )
    def cooc(left_h, right_h, weight_h, out_h, hist, lbuf, rbuf, offb, valb):
        c = lax.axis_index("c")
        t = lax.axis_index("s")
        lane = lax.iota(jnp.int32, _L)

        def scan_window(s_lo, s_hi):
            n_t = (nchunk - 1 - t) // _NS + 1

            def chunk_body(i, _):
                base = (t + i * _NS) * _CH
                pltpu.sync_copy(left_h.at[pl.ds(base, _CH)], lbuf)
                pltpu.sync_copy(right_h.at[pl.ds(base, _CH)], rbuf)

                def sub_body(s, _):
                    def vec_body(j, _):
                        k = s * _FLUSH + j * _L
                        lv = lbuf[pl.ds(k, _L)]
                        rv = rbuf[pl.ds(k, _L)]
                        m = (lv >= s_lo) & (lv < s_hi)
                        off = (lv - s_lo) * v + rv
                        pad = t * 8192 + j * _L + lane
                        off = jnp.where(m, off, pad)
                        val = jnp.where(m, jnp.float32(1.0), jnp.float32(0.0))
                        offb[pl.ds(j * _L, _L)] = off
                        valb[pl.ds(j * _L, _L)] = val
                        return 0
                    lax.fori_loop(0, _FLUSH // _L, vec_body, 0)
                    pltpu.sync_copy(valb, hist.at[offb], add=True)
                    return 0
                lax.fori_loop(0, _CH // _FLUSH, sub_body, 0)
                return 0
            lax.fori_loop(0, n_t, chunk_body, 0)

        def uniform_pass(p, _):
            s_lo = (p * _NC + c) * r
            w0 = s_lo * v + t * share
            pltpu.sync_copy(weight_h.at[pl.ds(w0, share)],
                            hist.at[pl.ds(t * share, share)])
            plsc.subcore_barrier()
            scan_window(s_lo, s_lo + r)
            plsc.subcore_barrier()
            pltpu.sync_copy(hist.at[pl.ds(t * share, share)],
                            out_h.at[pl.ds(w0, share)])
            plsc.subcore_barrier()
            return 0

        lax.fori_loop(0, nu, uniform_pass, 0)

        if rt:
            share_t = rt * v // _NS
            s_lo = nu * _NC * r + c * rt
            w0 = s_lo * v + t * share_t
            pltpu.sync_copy(weight_h.at[pl.ds(w0, share_t)],
                            hist.at[pl.ds(t * share_t, share_t)])
            plsc.subcore_barrier()
            scan_window(s_lo, s_lo + rt)
            plsc.subcore_barrier()
            pltpu.sync_copy(hist.at[pl.ds(t * share_t, share_t)],
                            out_h.at[pl.ds(w0, share_t)])

    return cooc(left, right, weight.reshape(-1)).reshape(v, v)


def kernel(left, right, weight):
    left = left.astype(jnp.int32)
    right = right.astype(jnp.int32)
    return _impl(left, right, weight)


# speedup vs baseline: 1.0998x; 1.0998x over previous
import functools

import jax
import jax.numpy as jnp
from jax import lax
from jax.experimental import pallas as pl
from jax.experimental.pallas import tpu as pltpu
from jax.experimental.pallas import tpu_sc as plsc

_NC = 2
_NS = 16
_L = 16

_ROWS_PER_SC = 242
_CH = 3200
_FLUSH = 128


def _impl(left, right, weight, *, interpret=False):
    n_pairs = left.shape[0]
    v = weight.shape[0]
    r = min(_ROWS_PER_SC, v // _NC)
    nu = v // (_NC * r)
    rem = v - nu * _NC * r
    rt = rem // _NC
    assert rem % _NC == 0
    assert n_pairs % _CH == 0 and _CH % _FLUSH == 0 and _FLUSH % _L == 0
    nchunk = n_pairs // _CH
    share = r * v // _NS
    assert r * v % _NS == 0
    if rt:
        assert rt * v % _NS == 0

    mesh = plsc.VectorSubcoreMesh(
        core_axis_name="c", subcore_axis_name="s",
        num_cores=_NC, num_subcores=_NS)

    @functools.partial(
        pl.kernel,
        out_type=jax.ShapeDtypeStruct((v * v,), jnp.float32),
        mesh=mesh,
        scratch_types=[
            pltpu.VMEM_SHARED((r * v,), jnp.float32),
            pltpu.VMEM((_CH,), jnp.int32),
            pltpu.VMEM((_CH,), jnp.int32),
            pltpu.VMEM((_FLUSH,), jnp.int32),
            pltpu.VMEM((_FLUSH,), jnp.float32),
        ],
        interpret=interpret,
    )
    def cooc(left_h, right_h, weight_h, out_h, hist, lbuf, rbuf, offb, valb):
        c = lax.axis_index("c")
        t = lax.axis_index("s")
        lane = lax.iota(jnp.int32, _L)

        def scan_window(s_lo, s_hi):
            n_t = (nchunk - 1 - t) // _NS + 1

            def chunk_body(i, _):
                base = (t + i * _NS) * _CH
                pltpu.sync_copy(left_h.at[pl.ds(base, _CH)], lbuf)
                pltpu.sync_copy(right_h.at[pl.ds(base, _CH)], rbuf)

                def sub_body(s, _):
                    def vec_body(j, _):
                        k = s * _FLUSH + j * _L
                        lv = lbuf[pl.ds(k, _L)]
                        rv = rbuf[pl.ds(k, _L)]
                        m = (lv >= s_lo) & (lv < s_hi)
                        off = (lv - s_lo) * v + rv
                        pad = t * 8192 + j * _L + lane
                        off = jnp.where(m, off, pad)
                        val = jnp.where(m, jnp.float32(1.0), jnp.float32(0.0))
                        offb[pl.ds(j * _L, _L)] = off
                        valb[pl.ds(j * _L, _L)] = val
                        return 0
                    lax.fori_loop(0, _FLUSH // _L, vec_body, 0)
                    pltpu.sync_copy(valb, hist.at[offb], add=True)
                    return 0
                lax.fori_loop(0, _CH // _FLUSH, sub_body, 0)
                return 0
            lax.fori_loop(0, n_t, chunk_body, 0)

        def uniform_pass(p, _):
            s_lo = (p * _NC + c) * r
            w0 = s_lo * v + t * share
            pltpu.sync_copy(weight_h.at[pl.ds(w0, share)],
                            hist.at[pl.ds(t * share, share)])
            plsc.subcore_barrier()
            scan_window(s_lo, s_lo + r)
            plsc.subcore_barrier()
            pltpu.sync_copy(hist.at[pl.ds(t * share, share)],
                            out_h.at[pl.ds(w0, share)])
            plsc.subcore_barrier()
            return 0

        lax.fori_loop(0, nu, uniform_pass, 0)

        if rt:
            share_t = rt * v // _NS
            s_lo = nu * _NC * r + c * rt
            w0 = s_lo * v + t * share_t
            pltpu.sync_copy(weight_h.at[pl.ds(w0, share_t)],
                            hist.at[pl.ds(t * share_t, share_t)])
            plsc.subcore_barrier()
            scan_window(s_lo, s_lo + rt)
            plsc.subcore_barrier()
            pltpu.sync_copy(hist.at[pl.ds(t * share_t, share_t)],
                            out_h.at[pl.ds(w0, share_t)])

    return cooc(left, right, weight.reshape(-1)).reshape(v, v)


def kernel(left, right, weight):
    left = left.astype(jnp.int32)
    right = right.astype(jnp.int32)
    return _impl(left, right, weight)


# speedup vs baseline: 1.2080x; 1.0984x over previous
import functools

import jax
import jax.numpy as jnp
from jax import lax
from jax.experimental import pallas as pl
from jax.experimental.pallas import tpu as pltpu
from jax.experimental.pallas import tpu_sc as plsc

_NC = 2
_NS = 16
_L = 16

_ROWS_PER_SC = 242
_CH = 1600
_FLUSH = 80
_BI = _FLUSH // _L
_NBLK = _CH // _FLUSH


def _impl(left, right, weight, *, interpret=False):
    n_pairs = left.shape[0]
    v = weight.shape[0]
    r = min(_ROWS_PER_SC, v // _NC)
    nu = v // (_NC * r)
    rem = v - nu * _NC * r
    rt = rem // _NC
    assert rem % _NC == 0
    assert n_pairs % _CH == 0 and _CH % _FLUSH == 0 and _FLUSH % _L == 0
    nchunk = n_pairs // _CH
    ntp = -(-nchunk // _NS)
    ntp += ntp % 2
    share = r * v // _NS
    assert r * v % _NS == 0
    if rt:
        assert rt * v % _NS == 0

    mesh = plsc.VectorSubcoreMesh(
        core_axis_name="c", subcore_axis_name="s",
        num_cores=_NC, num_subcores=_NS)

    @functools.partial(
        pl.kernel,
        out_type=jax.ShapeDtypeStruct((v * v,), jnp.float32),
        mesh=mesh,
        scratch_types=[
            pltpu.VMEM_SHARED((r * v,), jnp.float32),
            pltpu.VMEM((_CH,), jnp.int32),
            pltpu.VMEM((_CH,), jnp.int32),
            pltpu.VMEM((_CH,), jnp.int32),
            pltpu.VMEM((_CH,), jnp.int32),
            pltpu.VMEM((_FLUSH,), jnp.int32),
            pltpu.VMEM((_FLUSH,), jnp.float32),
            pltpu.SemaphoreType.DMA,
            pltpu.SemaphoreType.DMA,
            pltpu.SemaphoreType.DMA,
            pltpu.SemaphoreType.DMA,
        ],
        interpret=interpret,
    )
    def cooc(left_h, right_h, weight_h, out_h, hist,
             lb0, lb1, rb0, rb1, offb, valb,
             sl0, sl1, sr0, sr1):
        lbb, rbb = (lb0, lb1), (rb0, rb1)
        lsem, rsem = (sl0, sl1), (sr0, sr1)
        c = lax.axis_index("c")
        t = lax.axis_index("s")
        lane = lax.iota(jnp.int32, _L)

        def start_pair(ci, b):
            base = jnp.minimum(t + ci * _NS, nchunk - 1) * _CH
            pltpu.async_copy(left_h.at[pl.ds(base, _CH)], lbb[b], lsem[b])
            pltpu.async_copy(right_h.at[pl.ds(base, _CH)], rbb[b], rsem[b])

        def wait_pair(b):
            pltpu.make_async_copy(left_h.at[pl.ds(0, _CH)], lbb[b],
                                  lsem[b]).wait()
            pltpu.make_async_copy(right_h.at[pl.ds(0, _CH)], rbb[b],
                                  rsem[b]).wait()

        def scan_window(s_lo, s_hi):
            start_pair(0, 0)

            def fill_and_fire(lbuf, rbuf, s, hi_eff):
                for jj in range(_BI):
                    k = (s * _BI + jj) * _L
                    lv = lbuf[pl.ds(k, _L)]
                    rv = rbuf[pl.ds(k, _L)]
                    d = lv - s_lo
                    m = (d >= 0) & (lv < hi_eff)
                    pad = t * 4096 + jj * _L + lane
                    offb[pl.ds(jj * _L, _L)] = jnp.where(
                        m, d * v + rv, pad)
                    valb[pl.ds(jj * _L, _L)] = jnp.where(
                        m, jnp.float32(1.0), jnp.float32(0.0))
                pltpu.sync_copy(valb, hist.at[offb], add=True)

            def chunk_pair_body(i2, _):
                for b in (0, 1):
                    ci = i2 * 2 + b
                    wait_pair(b)

                    @pl.when(ci < ntp - 1)
                    def _():
                        start_pair(ci + 1, 1 - b)

                    live_i = ((t + ci * _NS) < nchunk).astype(jnp.int32)
                    hi_eff = s_lo + (s_hi - s_lo) * live_i
                    lbuf, rbuf = lbb[b], rbb[b]

                    def blk(s, _):
                        fill_and_fire(lbuf, rbuf, s, hi_eff)
                        return 0
                    lax.fori_loop(0, _NBLK, blk, 0)
                return 0
            lax.fori_loop(0, ntp // 2, chunk_pair_body, 0)

        def uniform_pass(p, _):
            s_lo = (p * _NC + c) * r
            w0 = s_lo * v + t * share
            pltpu.sync_copy(weight_h.at[pl.ds(w0, share)],
                            hist.at[pl.ds(t * share, share)])
            plsc.subcore_barrier()
            scan_window(s_lo, s_lo + r)
            plsc.subcore_barrier()
            pltpu.sync_copy(hist.at[pl.ds(t * share, share)],
                            out_h.at[pl.ds(w0, share)])
            plsc.subcore_barrier()
            return 0

        lax.fori_loop(0, nu, uniform_pass, 0)

        if rt:
            share_t = rt * v // _NS
            s_lo = nu * _NC * r + c * rt
            w0 = s_lo * v + t * share_t
            pltpu.sync_copy(weight_h.at[pl.ds(w0, share_t)],
                            hist.at[pl.ds(t * share_t, share_t)])
            plsc.subcore_barrier()
            scan_window(s_lo, s_lo + rt)
            plsc.subcore_barrier()
            pltpu.sync_copy(hist.at[pl.ds(t * share_t, share_t)],
                            out_h.at[pl.ds(w0, share_t)])

    return cooc(left, right, weight.reshape(-1)).reshape(v, v)


def kernel(left, right, weight):
    left = left.astype(jnp.int32)
    right = right.astype(jnp.int32)
    return _impl(left, right, weight)


# speedup vs baseline: 1.9700x; 1.6308x over previous
import functools

import jax
import jax.numpy as jnp
from jax import lax
from jax.experimental import pallas as pl
from jax.experimental.pallas import tpu as pltpu
from jax.experimental.pallas import tpu_sc as plsc

_NC = 2
_NS = 16
_L = 16

_ROWS_PER_SC = 236
_CH = 1600
_FLUSH = 1600
_BI = _FLUSH // _L
_NBLK = _CH // _FLUSH


def _impl(left, right, weight, *, interpret=False):
    n_pairs = left.shape[0]
    v = weight.shape[0]
    r = min(_ROWS_PER_SC, v // _NC)
    nu = v // (_NC * r)
    rem = v - nu * _NC * r
    rt = rem // _NC
    assert rem % _NC == 0
    assert n_pairs % _CH == 0 and _CH % _FLUSH == 0 and _FLUSH % _L == 0
    nchunk = n_pairs // _CH
    ntp = -(-nchunk // _NS)
    ntp += ntp % 2
    share = r * v // _NS
    assert r * v % _NS == 0
    if rt:
        assert rt * v % _NS == 0

    mesh = plsc.VectorSubcoreMesh(
        core_axis_name="c", subcore_axis_name="s",
        num_cores=_NC, num_subcores=_NS)

    @functools.partial(
        pl.kernel,
        out_type=jax.ShapeDtypeStruct((v * v,), jnp.float32),
        mesh=mesh,
        scratch_types=[
            pltpu.VMEM_SHARED((r * v,), jnp.float32),
            pltpu.VMEM((_CH,), jnp.int32),
            pltpu.VMEM((_CH,), jnp.int32),
            pltpu.VMEM((_CH,), jnp.int32),
            pltpu.VMEM((_CH,), jnp.int32),
            pltpu.VMEM((_FLUSH,), jnp.int32),
            pltpu.VMEM((_FLUSH,), jnp.float32),
            pltpu.SemaphoreType.DMA,
            pltpu.SemaphoreType.DMA,
            pltpu.SemaphoreType.DMA,
            pltpu.SemaphoreType.DMA,
        ],
        interpret=interpret,
    )
    def cooc(left_h, right_h, weight_h, out_h, hist,
             lb0, lb1, rb0, rb1, offb, valb,
             sl0, sl1, sr0, sr1):
        lbb, rbb = (lb0, lb1), (rb0, rb1)
        lsem, rsem = (sl0, sl1), (sr0, sr1)
        c = lax.axis_index("c")
        t = lax.axis_index("s")
        lane = lax.iota(jnp.int32, _L)

        def start_pair(ci, b):
            base = jnp.minimum(t + ci * _NS, nchunk - 1) * _CH
            pltpu.async_copy(left_h.at[pl.ds(base, _CH)], lbb[b], lsem[b])
            pltpu.async_copy(right_h.at[pl.ds(base, _CH)], rbb[b], rsem[b])

        def wait_pair(b):
            pltpu.make_async_copy(left_h.at[pl.ds(0, _CH)], lbb[b],
                                  lsem[b]).wait()
            pltpu.make_async_copy(right_h.at[pl.ds(0, _CH)], rbb[b],
                                  rsem[b]).wait()

        def scan_window(s_lo, s_hi):
            start_pair(0, 0)

            def fill_and_fire(lbuf, rbuf, s, hi_eff):
                def fill(jj, _):
                    k = (s * _BI + jj) * _L
                    lv = lbuf[pl.ds(k, _L)]
                    rv = rbuf[pl.ds(k, _L)]
                    d = lv - s_lo
                    m = (d >= 0) & (lv < hi_eff)
                    pad = t * 4096 + (jj % 64) * _L + lane
                    offb[pl.ds(jj * _L, _L)] = jnp.where(
                        m, d * v + rv, pad)
                    valb[pl.ds(jj * _L, _L)] = jnp.where(
                        m, jnp.float32(1.0), jnp.float32(0.0))
                    return 0
                lax.fori_loop(0, _BI, fill, 0)
                pltpu.sync_copy(valb, hist.at[offb], add=True)

            def chunk_pair_body(i2, _):
                for b in (0, 1):
                    ci = i2 * 2 + b
                    wait_pair(b)

                    @pl.when(ci < ntp - 1)
                    def _():
                        start_pair(ci + 1, 1 - b)

                    live_i = ((t + ci * _NS) < nchunk).astype(jnp.int32)
                    hi_eff = s_lo + (s_hi - s_lo) * live_i
                    lbuf, rbuf = lbb[b], rbb[b]

                    def blk(s, _):
                        fill_and_fire(lbuf, rbuf, s, hi_eff)
                        return 0
                    lax.fori_loop(0, _NBLK, blk, 0)
                return 0
            lax.fori_loop(0, ntp // 2, chunk_pair_body, 0)

        def uniform_pass(p, _):
            s_lo = (p * _NC + c) * r
            w0 = s_lo * v + t * share
            pltpu.sync_copy(weight_h.at[pl.ds(w0, share)],
                            hist.at[pl.ds(t * share, share)])
            plsc.subcore_barrier()
            scan_window(s_lo, s_lo + r)
            plsc.subcore_barrier()
            pltpu.sync_copy(hist.at[pl.ds(t * share, share)],
                            out_h.at[pl.ds(w0, share)])
            plsc.subcore_barrier()
            return 0

        lax.fori_loop(0, nu, uniform_pass, 0)

        if rt:
            share_t = rt * v // _NS
            s_lo = nu * _NC * r + c * rt
            w0 = s_lo * v + t * share_t
            pltpu.sync_copy(weight_h.at[pl.ds(w0, share_t)],
                            hist.at[pl.ds(t * share_t, share_t)])
            plsc.subcore_barrier()
            scan_window(s_lo, s_lo + rt)
            plsc.subcore_barrier()
            pltpu.sync_copy(hist.at[pl.ds(t * share_t, share_t)],
                            out_h.at[pl.ds(w0, share_t)])

    return cooc(left, right, weight.reshape(-1)).reshape(v, v)


def kernel(left, right, weight):
    left = left.astype(jnp.int32)
    right = right.astype(jnp.int32)
    return _impl(left, right, weight)
